# baseline (speedup 1.0000x reference)
import functools

import jax
import jax.numpy as jnp
from jax.experimental import pallas as pl

_NUM_HEADS = 16
_EPS = 1e-5
_LOG2E = 1.4426950408889634
_CLAMP = 43.0


def _ln(x, g, b):
    mu = jnp.mean(x, axis=-1, keepdims=True)
    ms = jnp.mean(x * x, axis=-1, keepdims=True)
    var = ms - mu * mu
    return (x - mu) * jax.lax.rsqrt(var + _EPS) * g + b


def _ln_qkv_body(x_ref, g_ref, b_ref, w_ref, vb_ref, q_ref, kt_ref, v_ref, *, d):
    h = _ln(x_ref[...], g_ref[...], b_ref[...]).astype(jnp.bfloat16)
    r = jnp.dot(h, w_ref[...], preferred_element_type=jnp.float32)
    q_ref[...] = r[:, :d].astype(jnp.bfloat16)
    kt_ref[0] = r[:, d:2 * d].astype(jnp.bfloat16).T
    v_ref[...] = (r[:, 2 * d:] + vb_ref[...]).astype(jnp.bfloat16)


def _attn_ffn_body(q_ref, kt_ref, v_ref, m_ref, x_ref, wo_ref, g2_ref,
                   b2g_ref, w1_ref, b1_ref, w2_ref, b2_ref, y_ref,
                   *, dh, scale):
    num_heads = q_ref.shape[-1] // dh
    mask = m_ref[0].astype(jnp.bfloat16)
    outs = []
    for h in range(num_heads):
        qh = q_ref[0, :, h * dh:(h + 1) * dh] * jnp.bfloat16(scale)
        kth = kt_ref[0, h * dh:(h + 1) * dh, :]
        vh = v_ref[0, :, h * 2 * dh:(h + 1) * 2 * dh]
        s = jnp.dot(qh, kth,
                    preferred_element_type=jnp.float32).astype(jnp.bfloat16)
        sb = jnp.minimum(s + mask, jnp.bfloat16(_CLAMP))
        p = jnp.exp2(sb)
        ol = jnp.dot(p, vh, preferred_element_type=jnp.float32)
        l = jnp.maximum(ol[:, dh:], 1e-30)
        outs.append(ol[:, :dh] / l)
    o = jnp.concatenate(outs, axis=-1).astype(jnp.bfloat16)
    xo = x_ref[0] + jnp.dot(o, wo_ref[...], preferred_element_type=jnp.float32)
    h2 = _ln(xo, g2_ref[...], b2g_ref[...]).astype(jnp.bfloat16)
    a = jnp.dot(h2, w1_ref[...], preferred_element_type=jnp.float32)
    g = jax.nn.gelu(a + b1_ref[...]).astype(jnp.bfloat16)
    m = jnp.dot(g, w2_ref[...], preferred_element_type=jnp.float32)
    y_ref[0] = xo + m + b2_ref[...]


def kernel(x, adj_mask, Wq, Wk, Wv, Wo, ln1_g, ln1_b, ln2_g, ln2_b,
           W1, b1, W2, b2):
    B, S, D = x.shape
    H = _NUM_HEADS
    DH = D // H
    FF = W1.shape[1]

    bq1 = min(512, S)
    bq2 = min(256, S)

    wv_aug = jnp.concatenate(
        [Wv.reshape(D, H, DH), jnp.zeros((D, H, DH), jnp.float32)],
        axis=-1).reshape(D, 2 * D)
    v_bias = jnp.concatenate(
        [jnp.zeros((1, H, DH), jnp.float32), jnp.ones((1, H, DH), jnp.float32)],
        axis=-1).reshape(1, 2 * D)
    w_all = jnp.concatenate([Wq, Wk, wv_aug], axis=1).astype(jnp.bfloat16)

    x3 = x
    x2 = x.reshape(B * S, D)
    g1 = ln1_g.reshape(1, D)
    bb1 = ln1_b.reshape(1, D)
    g2 = ln2_g.reshape(1, D)
    bb2 = ln2_b.reshape(1, D)
    bias1 = b1.reshape(1, FF)
    bias2 = b2.reshape(1, D)
    mask_add = adj_mask.reshape(B, S, S)

    nq1 = S // bq1
    rows = pl.BlockSpec((bq1, D), lambda i: (i, 0))
    rows2 = pl.BlockSpec((bq1, 2 * D), lambda i: (i, 0))
    ktspec = pl.BlockSpec((1, D, bq1), lambda i: (i // nq1, 0, i % nq1))
    q2, kt, v2 = pl.pallas_call(
        functools.partial(_ln_qkv_body, d=D),
        grid=(B * S // bq1,),
        in_specs=[rows,
                  pl.BlockSpec((1, D), lambda i: (0, 0)),
                  pl.BlockSpec((1, D), lambda i: (0, 0)),
                  pl.BlockSpec((D, 4 * D), lambda i: (0, 0)),
                  pl.BlockSpec((1, 2 * D), lambda i: (0, 0))],
        out_specs=[rows, ktspec, rows2],
        out_shape=[jax.ShapeDtypeStruct((B * S, D), jnp.bfloat16),
                   jax.ShapeDtypeStruct((B, D, S), jnp.bfloat16),
                   jax.ShapeDtypeStruct((B * S, 2 * D), jnp.bfloat16)],
    )(x2, g1, bb1, w_all, v_bias)

    q3 = q2.reshape(B, S, D)
    v3 = v2.reshape(B, S, 2 * D)

    scale = _LOG2E / (DH ** 0.5)
    qspec = pl.BlockSpec((1, bq2, D), lambda b, i: (b, i, 0))
    ktspec2 = pl.BlockSpec((1, D, S), lambda b, i: (b, 0, 0))
    vspec = pl.BlockSpec((1, S, 2 * D), lambda b, i: (b, 0, 0))
    mspec = pl.BlockSpec((1, bq2, S), lambda b, i: (b, i, 0))
    const = lambda shape: pl.BlockSpec(shape, lambda b, i: tuple(0 for _ in shape))
    y = pl.pallas_call(
        functools.partial(_attn_ffn_body, dh=DH, scale=scale),
        grid=(B, S // bq2),
        in_specs=[qspec, ktspec2, vspec, mspec, qspec,
                  const((D, D)), const((1, D)), const((1, D)),
                  const((D, FF)), const((1, FF)),
                  const((FF, D)), const((1, D))],
        out_specs=qspec,
        out_shape=jax.ShapeDtypeStruct((B, S, D), jnp.float32),
    )(q3, kt, v3, mask_add, x3, Wo.astype(jnp.bfloat16), g2, bb2,
      W1.astype(jnp.bfloat16), bias1, W2.astype(jnp.bfloat16), bias2)

    return y


# speedup vs baseline: 1.0735x; 1.0735x over previous
import functools

import jax
import jax.numpy as jnp
from jax.experimental import pallas as pl
from jax.experimental.pallas import tpu as pltpu

_NUM_HEADS = 16
_EPS = 1e-5
_LOG2E = 1.4426950408889634
_CLAMP = 43.0


def _ln(x, g, b):
    mu = jnp.mean(x, axis=-1, keepdims=True)
    ms = jnp.mean(x * x, axis=-1, keepdims=True)
    var = ms - mu * mu
    return (x - mu) * jax.lax.rsqrt(var + _EPS) * g + b


def _ln_qkv_body(x_ref, g_ref, b_ref, w_ref, vb_ref, q_ref, kt_ref, v_ref, *, d):
    h = _ln(x_ref[...], g_ref[...], b_ref[...]).astype(jnp.bfloat16)
    r = jnp.dot(h, w_ref[...], preferred_element_type=jnp.float32)
    q_ref[...] = r[:, :d].astype(jnp.bfloat16)
    kt_ref[0] = r[:, d:2 * d].astype(jnp.bfloat16).T
    v_ref[...] = (r[:, 2 * d:] + vb_ref[...]).astype(jnp.bfloat16)


def _attn_ffn_body(q_ref, kt_ref, v_ref, m_ref, x_ref, wo_ref, g2_ref,
                   b2g_ref, w1_ref, b1_ref, w2_ref, b2_ref, y_ref,
                   *, dh, scale):
    num_heads = q_ref.shape[-1] // dh
    mask = m_ref[0].astype(jnp.bfloat16)
    outs = []
    for h in range(num_heads):
        qh = q_ref[0, :, h * dh:(h + 1) * dh] * jnp.bfloat16(scale)
        kth = kt_ref[0, h * dh:(h + 1) * dh, :]
        vh = v_ref[0, :, h * 2 * dh:(h + 1) * 2 * dh]
        s = jnp.dot(qh, kth,
                    preferred_element_type=jnp.float32).astype(jnp.bfloat16)
        sb = jnp.minimum(s + mask, jnp.bfloat16(_CLAMP))
        p = jnp.exp2(sb)
        ol = jnp.dot(p, vh, preferred_element_type=jnp.float32)
        l = jnp.maximum(ol[:, dh:], 1e-30)
        outs.append(ol[:, :dh] / l)
    o = jnp.concatenate(outs, axis=-1).astype(jnp.bfloat16)
    xo = x_ref[0] + jnp.dot(o, wo_ref[...], preferred_element_type=jnp.float32)
    h2 = _ln(xo, g2_ref[...], b2g_ref[...]).astype(jnp.bfloat16)
    a = jnp.dot(h2, w1_ref[...], preferred_element_type=jnp.float32)
    g = jax.nn.gelu(a + b1_ref[...]).astype(jnp.bfloat16)
    m = jnp.dot(g, w2_ref[...], preferred_element_type=jnp.float32)
    y_ref[0] = xo + m + b2_ref[...]


def kernel(x, adj_mask, Wq, Wk, Wv, Wo, ln1_g, ln1_b, ln2_g, ln2_b,
           W1, b1, W2, b2):
    B, S, D = x.shape
    H = _NUM_HEADS
    DH = D // H
    FF = W1.shape[1]

    bq1 = min(512, S)
    bq2 = min(512, S)

    wv_aug = jnp.concatenate(
        [Wv.reshape(D, H, DH), jnp.zeros((D, H, DH), jnp.float32)],
        axis=-1).reshape(D, 2 * D)
    v_bias = jnp.concatenate(
        [jnp.zeros((1, H, DH), jnp.float32), jnp.ones((1, H, DH), jnp.float32)],
        axis=-1).reshape(1, 2 * D)
    w_all = jnp.concatenate([Wq, Wk, wv_aug], axis=1).astype(jnp.bfloat16)

    x3 = x
    x2 = x.reshape(B * S, D)
    g1 = ln1_g.reshape(1, D)
    bb1 = ln1_b.reshape(1, D)
    g2 = ln2_g.reshape(1, D)
    bb2 = ln2_b.reshape(1, D)
    bias1 = b1.reshape(1, FF)
    bias2 = b2.reshape(1, D)
    mask_add = adj_mask.reshape(B, S, S)

    nq1 = S // bq1
    rows = pl.BlockSpec((bq1, D), lambda i: (i, 0))
    rows2 = pl.BlockSpec((bq1, 2 * D), lambda i: (i, 0))
    ktspec = pl.BlockSpec((1, D, bq1), lambda i: (i // nq1, 0, i % nq1))
    q2, kt, v2 = pl.pallas_call(
        functools.partial(_ln_qkv_body, d=D),
        grid=(B * S // bq1,),
        in_specs=[rows,
                  pl.BlockSpec((1, D), lambda i: (0, 0)),
                  pl.BlockSpec((1, D), lambda i: (0, 0)),
                  pl.BlockSpec((D, 4 * D), lambda i: (0, 0)),
                  pl.BlockSpec((1, 2 * D), lambda i: (0, 0))],
        out_specs=[rows, ktspec, rows2],
        out_shape=[jax.ShapeDtypeStruct((B * S, D), jnp.bfloat16),
                   jax.ShapeDtypeStruct((B, D, S), jnp.bfloat16),
                   jax.ShapeDtypeStruct((B * S, 2 * D), jnp.bfloat16)],
    )(x2, g1, bb1, w_all, v_bias)

    q3 = q2.reshape(B, S, D)
    v3 = v2.reshape(B, S, 2 * D)

    scale = _LOG2E / (DH ** 0.5)
    qspec = pl.BlockSpec((1, bq2, D), lambda b, i: (b, i, 0))
    ktspec2 = pl.BlockSpec((1, D, S), lambda b, i: (b, 0, 0))
    vspec = pl.BlockSpec((1, S, 2 * D), lambda b, i: (b, 0, 0))
    mspec = pl.BlockSpec((1, bq2, S), lambda b, i: (b, i, 0))
    const = lambda shape: pl.BlockSpec(shape, lambda b, i: tuple(0 for _ in shape))
    y = pl.pallas_call(
        functools.partial(_attn_ffn_body, dh=DH, scale=scale),
        grid=(B, S // bq2),
        in_specs=[qspec, ktspec2, vspec, mspec, qspec,
                  const((D, D)), const((1, D)), const((1, D)),
                  const((D, FF)), const((1, FF)),
                  const((FF, D)), const((1, D))],
        out_specs=qspec,
        out_shape=jax.ShapeDtypeStruct((B, S, D), jnp.float32),
        compiler_params=pltpu.CompilerParams(
            vmem_limit_bytes=100 * 1024 * 1024),
    )(q3, kt, v3, mask_add, x3, Wo.astype(jnp.bfloat16), g2, bb2,
      W1.astype(jnp.bfloat16), bias1, W2.astype(jnp.bfloat16), bias2)

    return y


# speedup vs baseline: 1.0896x; 1.0151x over previous
import functools

import jax
import jax.numpy as jnp
from jax.experimental import pallas as pl
from jax.experimental.pallas import tpu as pltpu

_NUM_HEADS = 16
_EPS = 1e-5
_LOG2E = 1.4426950408889634
_CLAMP = 43.0


def _ln(x, g, b):
    mu = jnp.mean(x, axis=-1, keepdims=True)
    ms = jnp.mean(x * x, axis=-1, keepdims=True)
    var = ms - mu * mu
    return (x - mu) * jax.lax.rsqrt(var + _EPS) * g + b


def _fused_body(x_ref, m_ref, w_ref, vb_ref, g1_ref, b1g_ref, wo_ref,
                g2_ref, b2g_ref, w1_ref, bias1_ref, w2_ref, bias2_ref,
                y_ref, q_scr, kt_scr, v_scr, *, d, dh, bq, nq):
    i = pl.program_id(1)

    @pl.when(i < nq)
    def _qkv():
        h = _ln(x_ref[0], g1_ref[...], b1g_ref[...]).astype(jnp.bfloat16)
        r = jnp.dot(h, w_ref[...], preferred_element_type=jnp.float32)
        rows = pl.ds(i * bq, bq)
        q_scr[rows, :] = r[:, :d].astype(jnp.bfloat16)
        kt_scr[:, rows] = r[:, d:2 * d].astype(jnp.bfloat16).T
        v_scr[rows, :] = (r[:, 2 * d:] + vb_ref[...]).astype(jnp.bfloat16)

    @pl.when(i >= nq)
    def _attn_ffn():
        num_heads = d // dh
        mask = m_ref[0].astype(jnp.bfloat16)
        q_rows = q_scr[pl.ds((i - nq) * bq, bq), :]
        outs = []
        for h in range(num_heads):
            qh = q_rows[:, h * dh:(h + 1) * dh]
            kth = kt_scr[h * dh:(h + 1) * dh, :]
            vh = v_scr[:, h * 2 * dh:(h + 1) * 2 * dh]
            s = jnp.dot(qh, kth,
                        preferred_element_type=jnp.float32).astype(jnp.bfloat16)
            sb = jnp.minimum(s + mask, jnp.bfloat16(_CLAMP))
            p = jnp.exp2(sb)
            ol = jnp.dot(p, vh, preferred_element_type=jnp.float32)
            l = jnp.maximum(ol[:, dh:], 1e-30)
            outs.append(ol[:, :dh] / l)
        o = jnp.concatenate(outs, axis=-1).astype(jnp.bfloat16)
        xo = x_ref[0] + jnp.dot(o, wo_ref[...],
                                preferred_element_type=jnp.float32)
        h2 = _ln(xo, g2_ref[...], b2g_ref[...]).astype(jnp.bfloat16)
        a = jnp.dot(h2, w1_ref[...], preferred_element_type=jnp.float32)
        g = jax.nn.gelu(a + bias1_ref[...]).astype(jnp.bfloat16)
        m = jnp.dot(g, w2_ref[...], preferred_element_type=jnp.float32)
        y_ref[0] = xo + m + bias2_ref[...]


def kernel(x, adj_mask, Wq, Wk, Wv, Wo, ln1_g, ln1_b, ln2_g, ln2_b,
           W1, b1, W2, b2):
    B, S, D = x.shape
    H = _NUM_HEADS
    DH = D // H
    FF = W1.shape[1]

    bq = min(512, S)
    nq = S // bq

    scale = _LOG2E / (DH ** 0.5)
    wv_aug = jnp.concatenate(
        [Wv.reshape(D, H, DH), jnp.zeros((D, H, DH), jnp.float32)],
        axis=-1).reshape(D, 2 * D)
    v_bias = jnp.concatenate(
        [jnp.zeros((1, H, DH), jnp.float32), jnp.ones((1, H, DH), jnp.float32)],
        axis=-1).reshape(1, 2 * D)
    w_all = jnp.concatenate([Wq * scale, Wk, wv_aug],
                            axis=1).astype(jnp.bfloat16)

    g1 = ln1_g.reshape(1, D)
    bb1 = ln1_b.reshape(1, D)
    g2 = ln2_g.reshape(1, D)
    bb2 = ln2_b.reshape(1, D)
    bias1 = b1.reshape(1, FF)
    bias2 = b2.reshape(1, D)
    mask_add = adj_mask.reshape(B, S, S)

    xspec = pl.BlockSpec(
        (1, bq, D), lambda b, i: (b, jnp.where(i < nq, i, i - nq), 0))
    mspec = pl.BlockSpec(
        (1, bq, S), lambda b, i: (b, jnp.maximum(i - nq, 0), 0))
    yspec = pl.BlockSpec(
        (1, bq, D), lambda b, i: (b, jnp.maximum(i - nq, 0), 0))
    const = lambda shape: pl.BlockSpec(shape, lambda b, i: tuple(0 for _ in shape))
    y = pl.pallas_call(
        functools.partial(_fused_body, d=D, dh=DH, bq=bq, nq=nq),
        grid=(B, 2 * nq),
        in_specs=[xspec, mspec,
                  const((D, 4 * D)), const((1, 2 * D)),
                  const((1, D)), const((1, D)),
                  const((D, D)), const((1, D)), const((1, D)),
                  const((D, FF)), const((1, FF)),
                  const((FF, D)), const((1, D))],
        out_specs=yspec,
        out_shape=jax.ShapeDtypeStruct((B, S, D), jnp.float32),
        scratch_shapes=[pltpu.VMEM((S, D), jnp.bfloat16),
                        pltpu.VMEM((D, S), jnp.bfloat16),
                        pltpu.VMEM((S, 2 * D), jnp.bfloat16)],
        compiler_params=pltpu.CompilerParams(
            vmem_limit_bytes=100 * 1024 * 1024),
    )(x, mask_add, w_all, v_bias, g1, bb1, Wo.astype(jnp.bfloat16), g2, bb2,
      W1.astype(jnp.bfloat16), bias1, W2.astype(jnp.bfloat16), bias2)

    return y
